# baseline (speedup 1.0000x reference)
import functools
import jax
import jax.numpy as jnp
from jax import lax
from jax.experimental import pallas as pl
from jax.experimental.pallas import tpu as pltpu
from jax.experimental.pallas import tpu_sc as plsc

_BLK = 4096
_L = 16


def _sc_select(p0, p1, p2, idx):
    info = plsc.get_sparse_core_info()
    nw = info.num_cores * info.num_subcores
    b = idx.shape[0]
    bpw = b // nw

    mesh = plsc.VectorSubcoreMesh(core_axis_name="c", subcore_axis_name="s")

    @functools.partial(
        pl.kernel,
        mesh=mesh,
        out_type=jax.ShapeDtypeStruct((b,), jnp.float32),
        scratch_types=[
            pltpu.VMEM((bpw,), jnp.float32),
            pltpu.VMEM((bpw,), jnp.float32),
            pltpu.VMEM((bpw,), jnp.float32),
            pltpu.VMEM((bpw,), jnp.int32),
            pltpu.VMEM((bpw,), jnp.float32),
        ],
        compiler_params=pltpu.CompilerParams(needs_layout_passes=False),
    )
    def k(p0_hbm, p1_hbm, p2_hbm, idx_hbm, out_hbm,
          p0_v, p1_v, p2_v, idx_v, out_v):
        wid = lax.axis_index("s") * info.num_cores + lax.axis_index("c")
        base = wid * bpw
        pltpu.sync_copy(p0_hbm.at[pl.ds(base, bpw)], p0_v)
        pltpu.sync_copy(p1_hbm.at[pl.ds(base, bpw)], p1_v)
        pltpu.sync_copy(p2_hbm.at[pl.ds(base, bpw)], p2_v)
        pltpu.sync_copy(idx_hbm.at[pl.ds(base, bpw)], idx_v)
        for j in range(bpw // _L):
            s = pl.ds(j * _L, _L)
            iv = idx_v[s]
            sel = jnp.where(iv == 0, p0_v[s],
                            jnp.where(iv == 1, p1_v[s], p2_v[s]))
            out_v[s] = sel
        pltpu.sync_copy(out_v, out_hbm.at[pl.ds(base, bpw)])

    return k(p0, p1, p2, idx)


def _tc_body(x_ref, w1_ref, b1_ref, w2_ref, b2_ref, w3_ref, emb_ref,
             b3_ref, w4_ref, b4_ref, out_ref, out1_ref, out2_ref):
    f = jnp.maximum(x_ref[...] @ w1_ref[...] + b1_ref[...], 0.0)
    f = jnp.maximum(f @ w2_ref[...] + b2_ref[...], 0.0)
    outs = (out_ref, out1_ref, out2_ref)
    for i in range(3):
        t = emb_ref[i:i + 1, :] @ w3_ref[i, 32:, :] + b3_ref[i:i + 1, :]
        h = jnp.maximum(f @ w3_ref[i, :32, :] + t, 0.0)
        row = lax.dot_general(w4_ref[i], h, (((0,), (1,)), ((), ())))
        outs[i][...] = row.reshape(-1) + b4_ref[i, 0]


def _tc_call(x, w1, b1r, w2, b2r, w3, emb, b3, w4, b4):
    bsz = x.shape[0]
    full = lambda i: (0, 0)
    full3 = lambda i: (0, 0, 0)
    return pl.pallas_call(
        _tc_body,
        grid=(bsz // _BLK,),
        in_specs=[
            pl.BlockSpec((_BLK, 128), lambda i: (i, 0)),
            pl.BlockSpec((128, 64), full),
            pl.BlockSpec((1, 64), full),
            pl.BlockSpec((64, 32), full),
            pl.BlockSpec((1, 32), full),
            pl.BlockSpec((3, 48, 64), full3),
            pl.BlockSpec((3, 16), full),
            pl.BlockSpec((3, 64), full),
            pl.BlockSpec((3, 64, 1), full3),
            pl.BlockSpec((3, 1), full),
        ],
        out_specs=[pl.BlockSpec((_BLK,), lambda i: (i,))] * 3,
        out_shape=[jax.ShapeDtypeStruct((bsz,), jnp.float32)] * 3,
        compiler_params=pltpu.CompilerParams(
            dimension_semantics=("arbitrary",)),
    )(x, w1, b1r, w2, b2r, w3, emb, b3, w4, b4)


def kernel(x, regime_ids, W1, b1, W2, b2, emb, W3, b3, W4, b4):
    idx = regime_ids.astype(jnp.int32)
    p0, p1, p2 = _tc_call(x, W1, b1.reshape(1, -1), W2, b2.reshape(1, -1),
                          W3, emb, b3, W4, b4)
    return _sc_select(p0, p1, p2, idx).reshape(-1, 1)
